# baseline (speedup 1.0000x reference)
import functools

import jax
import jax.numpy as jnp
from jax import lax
from jax.experimental import pallas as pl
from jax.experimental.pallas import tpu as pltpu
from jax.experimental.pallas import tpu_sc as plsc

_B = 16384
_D = 64
_NEG = 5
_NC = 2
_NS = 16
_NW = _NC * _NS
_L = 16
_BPW = _B // _NW
_C = 64
_NCHUNK = _BPW // _C
_GROUPS = _C // _L
_OC = 8
_AP = 1000064


def _sc_scores_body(U2, V2, R2, A2, pos_u, pos_v, negv_t, pos_r, out,
                    idx_u, idx_v, idx_r, idx_n0, idx_n1, idx_n2, idx_n3,
                    idx_n4, row_a, rows_u, rows_v, rows_r, rows_a,
                    rows_n0, rows_n1, rows_n2, rows_n3, rows_n4,
                    out_buf, sem):
    idx_n = [idx_n0, idx_n1, idx_n2, idx_n3, idx_n4]
    rows_n = [rows_n0, rows_n1, rows_n2, rows_n3, rows_n4]

    wid = lax.axis_index("s") * _NC + lax.axis_index("c")
    ids0 = lax.iota(jnp.int32, _L)

    for c in range(_NCHUNK):
        base = pl.multiple_of(wid * _BPW + c * _C, _C)

        pltpu.sync_copy(pos_u.at[pl.ds(base, _C)], idx_u)
        pltpu.sync_copy(pos_v.at[pl.ds(base, _C)], idx_v)
        pltpu.sync_copy(pos_r.at[pl.ds(base, _C)], idx_r)
        for k in range(_NEG):
            pltpu.sync_copy(negv_t.at[pl.ds(k * _B + base, _C)], idx_n[k])

        for j in range(_GROUPS):
            sl = pl.ds(j * _L, _L)
            row_a[sl] = lax.shift_right_logical(idx_u[sl], 7)

        cps = [
            pltpu.async_copy(U2.at[idx_u], rows_u, sem),
            pltpu.async_copy(R2.at[idx_r], rows_r, sem),
            pltpu.async_copy(V2.at[idx_v], rows_v, sem),
            pltpu.async_copy(A2.at[row_a], rows_a, sem),
        ]
        for k in range(_NEG):
            cps.append(pltpu.async_copy(V2.at[idx_n[k]], rows_n[k], sem))
        for cp in cps:
            cp.wait()

        for g in range(_GROUPS):
            ids = ids0 + (g * _L)
            sl = pl.ds(g * _L, _L)
            lane_a = jnp.bitwise_and(idx_u[sl], 127)
            a = plsc.load_gather(rows_a, [ids, lane_a])
            a = jnp.minimum(jnp.maximum(a, 0.01), 0.99)

            def d_body(d, accs):
                dd = jnp.zeros((_L,), jnp.int32) + d
                u = plsc.load_gather(rows_u, [ids, dd])
                r = plsc.load_gather(rows_r, [ids, dd])
                v = plsc.load_gather(rows_v, [ids, dd])
                nxt = [accs[0] + u * v, accs[1] + r * v]
                for k in range(_NEG):
                    n = plsc.load_gather(rows_n[k], [ids, dd])
                    nxt.append(accs[2 + 2 * k] + u * n)
                    nxt.append(accs[3 + 2 * k] + r * n)
                return tuple(nxt)

            zero = jnp.zeros((_L,), jnp.float32)
            accs = lax.fori_loop(0, _D, d_body, (zero,) * (2 * (1 + _NEG)))

            oidx = ids * _OC
            s = a * accs[0] + (1.0 - a) * accs[1]
            s = jnp.minimum(jnp.maximum(s, -10.0), 10.0)
            plsc.store_scatter(out_buf, [oidx], s)
            for k in range(_NEG):
                s = a * accs[2 + 2 * k] + (1.0 - a) * accs[3 + 2 * k]
                s = jnp.minimum(jnp.maximum(s, -10.0), 10.0)
                plsc.store_scatter(out_buf, [oidx + (k + 1)], s)

        pltpu.sync_copy(out_buf, out.at[pl.ds(base * _OC, _C * _OC)])


_sc_scores = functools.partial(
    pl.kernel,
    out_type=jax.ShapeDtypeStruct((_B * _OC,), jnp.float32),
    mesh=plsc.VectorSubcoreMesh(core_axis_name="c", subcore_axis_name="s"),
    scratch_types=(
        [pltpu.VMEM((_C,), jnp.int32) for _ in range(8)]
        + [pltpu.VMEM((_C,), jnp.int32)]
        + [pltpu.VMEM((_C, 128), jnp.float32) for _ in range(9)]
        + [pltpu.VMEM((_C * _OC,), jnp.float32),
           pltpu.SemaphoreType.DMA]
    ),
    compiler_params=pltpu.CompilerParams(needs_layout_passes=False,
                                         use_tc_tiling_on_sc=True),
)(_sc_scores_body)


_TBLK = 8192


def _tpose_body(src_ref, dst_ref):
    x = src_ref[...]
    y = x.T
    dst_ref[...] = jnp.concatenate([y, y], axis=1)


def _tc_transpose(xt, n):
    grid = (n + _TBLK - 1) // _TBLK
    return pl.pallas_call(
        _tpose_body,
        grid=(grid,),
        in_specs=[pl.BlockSpec((_D, _TBLK), lambda b: (0, b))],
        out_specs=pl.BlockSpec((_TBLK, 128), lambda b: (b, 0)),
        out_shape=jax.ShapeDtypeStruct((n, 128), jnp.float32),
    )(xt)


def _loss_body(s_ref, o_ref):
    s = s_ref[...]
    cid = lax.broadcasted_iota(jnp.int32, s.shape, 1) % _OC
    x = jnp.where(cid == 0, -s, s)
    sp = jnp.maximum(x, 0.0) + jnp.log1p(jnp.exp(-jnp.abs(x)))
    sp = jnp.where(cid < 1 + _NEG, sp, 0.0)
    o_ref[...] = (jnp.sum(sp) * (1.0 / _B)).reshape(1, 1)


def kernel(U, V, R, alpha, pos_u, pos_v, neg_v, pos_r):
    pos_u = pos_u.astype(jnp.int32)
    pos_v = pos_v.astype(jnp.int32)
    pos_r = pos_r.astype(jnp.int32)
    negv_t = jnp.transpose(neg_v.astype(jnp.int32)).reshape(_NEG * _B)

    U2 = _tc_transpose(jnp.transpose(U), U.shape[0])
    V2 = _tc_transpose(jnp.transpose(V), V.shape[0])
    R2 = _tc_transpose(jnp.transpose(R), R.shape[0])
    A2 = jnp.pad(alpha, (0, _AP - alpha.shape[0])).reshape(_AP // 128, 128)

    scores = _sc_scores(U2, V2, R2, A2, pos_u, pos_v, negv_t, pos_r)
    scores2d = scores.reshape(_B * _OC // 128, 128)

    loss = pl.pallas_call(
        _loss_body,
        out_shape=jax.ShapeDtypeStruct((1, 1), jnp.float32),
    )(scores2d)
    return loss[0, 0]
